# baseline (speedup 1.0000x reference)
import functools
import jax
import jax.numpy as jnp
from jax import lax
from jax.experimental import pallas as pl
from jax.experimental.pallas import tpu as pltpu
from jax.experimental.pallas import tpu_sc as plsc

_N = 10000
_E = 320000
_D = 128
_HD = _D // 2
_L = 3
_BN_EPS = 1e-5

_NC = 2
_NS = 16
_CH = 128
_NB = 10
_CPB = 16
_CPT = _NB * _CPB
_EPT = _CPT * _CH
_EPAD = _NS * _EPT
_NPAD = 10112


def _sc_gather_segsum(h2, srcs, dsts, zeros):
    mesh = plsc.VectorSubcoreMesh(core_axis_name="c", subcore_axis_name="s")

    @functools.partial(
        pl.kernel,
        out_type=jax.ShapeDtypeStruct((_NC, _N, _HD), jnp.float32),
        mesh=mesh,
        scratch_types=[
            pltpu.VMEM_SHARED((_N, _HD), jnp.float32),
            pltpu.VMEM_SHARED((_NPAD, _HD), jnp.float32),
            pltpu.VMEM((2, _CPB, _CH), jnp.int32),
            pltpu.VMEM((2, _CPB, _CH), jnp.int32),
            pltpu.VMEM((2, _CH, _HD), jnp.float32),
            [pltpu.SemaphoreType.DMA] * 2,
            [pltpu.SemaphoreType.DMA] * 2,
            pltpu.SemaphoreType.DMA,
            pltpu.SemaphoreType.DMA,
        ],
    )
    def k(h2_hbm, srcs_hbm, dsts_hbm, zeros_hbm, out_hbm,
          h_s, agg_s, src_v, dst_v, rows_v, gsem, ssem, isem_s, isem_d):
        c = lax.axis_index("c")
        s = lax.axis_index("s")

        @pl.when(s < _NS - 1)
        def _():
            pltpu.sync_copy(h2_hbm.at[c, pl.ds(s * 624, 624)],
                            h_s.at[pl.ds(s * 624, 624)])

        @pl.when(s == _NS - 1)
        def _():
            pltpu.sync_copy(h2_hbm.at[c, pl.ds(9360, 640)],
                            h_s.at[pl.ds(9360, 640)])

        pltpu.sync_copy(zeros_hbm.at[pl.ds(s * 632, 632)],
                        agg_s.at[pl.ds(s * 632, 632)])
        pltpu.sync_copy(srcs_hbm.at[s, 0], src_v.at[0])
        pltpu.sync_copy(dsts_hbm.at[s, 0], dst_v.at[0])
        plsc.subcore_barrier()

        def fire_g(ksrc, j, q):
            pltpu.async_copy(h_s.at[ksrc.at[j]], rows_v.at[q], gsem[q])

        def wait_g(ksrc, j, q):
            pltpu.make_async_copy(h_s.at[ksrc.at[j]], rows_v.at[q],
                                  gsem[q]).wait()

        def fire_s(kdst, j, q):
            pltpu.async_copy(rows_v.at[q], agg_s.at[kdst.at[j]], ssem[q],
                             add=True)

        def wait_s(kdst, j, q):
            pltpu.make_async_copy(rows_v.at[q], agg_s.at[kdst.at[j]],
                                  ssem[q]).wait()

        for k in range(_NB):
            b = k % 2
            nb = (k + 1) % 2
            ksrc = src_v.at[b]
            kdst = dst_v.at[b]
            if k + 1 < _NB:
                pltpu.async_copy(srcs_hbm.at[s, k + 1], src_v.at[nb], isem_s)
                pltpu.async_copy(dsts_hbm.at[s, k + 1], dst_v.at[nb], isem_d)

            for q in range(2):
                fire_g(ksrc, q, q)

            def pair(t, _, ksrc=ksrc, kdst=kdst):
                for q in range(2):
                    wait_g(ksrc, 2 * t + q, q)
                    fire_s(kdst, 2 * t + q, q)
                for q in range(2):
                    wait_s(kdst, 2 * t + q, q)
                    fire_g(ksrc, 2 * t + q + 2, q)
                return 0

            lax.fori_loop(0, _CPB // 2 - 1, pair, 0)

            last = _CPB - 2
            for q in range(2):
                wait_g(ksrc, last + q, q)
                fire_s(kdst, last + q, q)
            for q in range(2):
                wait_s(kdst, last + q, q)

            if k + 1 < _NB:
                pltpu.make_async_copy(srcs_hbm.at[s, k + 1], src_v.at[nb],
                                      isem_s).wait()
                pltpu.make_async_copy(dsts_hbm.at[s, k + 1], dst_v.at[nb],
                                      isem_d).wait()
        plsc.subcore_barrier()

        @pl.when(s < _NS - 1)
        def _():
            pltpu.sync_copy(agg_s.at[pl.ds(s * 624, 624)],
                            out_hbm.at[c, pl.ds(s * 624, 624)])

        @pl.when(s == _NS - 1)
        def _():
            pltpu.sync_copy(agg_s.at[pl.ds(9360, 640)],
                            out_hbm.at[c, pl.ds(9360, 640)])

    return k(h2, srcs, dsts, zeros)


def _tc_mlp_bn(h, agg2, w1, b1, w2, b2, gm, bt):
    def body(h_ref, agg_ref, w1_ref, b1_ref, w2_ref, b2_ref, gm_ref, bt_ref,
             out_ref, out2_ref):
        z = h_ref[...] + jnp.concatenate([agg_ref[0], agg_ref[1]], axis=1)
        z = jnp.dot(z, w1_ref[...], preferred_element_type=jnp.float32)
        z = jnp.maximum(z + b1_ref[...], 0.0)
        z = jnp.dot(z, w2_ref[...], preferred_element_type=jnp.float32)
        z = jnp.maximum(z + b2_ref[...], 0.0)
        mean = jnp.mean(z, axis=0, keepdims=True)
        zc = z - mean
        var = jnp.mean(zc * zc, axis=0, keepdims=True)
        h_new = zc * lax.rsqrt(var + _BN_EPS) * gm_ref[...] + bt_ref[...]
        out_ref[...] = h_new
        out2_ref[0] = h_new[:, :_HD]
        out2_ref[1] = h_new[:, _HD:]

    return pl.pallas_call(
        body,
        out_shape=(jax.ShapeDtypeStruct((_N, _D), jnp.float32),
                   jax.ShapeDtypeStruct((_NC, _N, _HD), jnp.float32)),
    )(h, agg2, w1, b1, w2, b2, gm, bt)


def kernel(x, edge_index, batch, W1, b1, W2, b2, gamma, beta):
    src = edge_index[0]
    dst = edge_index[1]
    pad = _EPAD - _E
    srcs = jnp.concatenate([src, jnp.zeros((pad,), jnp.int32)]).reshape(
        _NS, _NB, _CPB, _CH)
    pad_dst = _N + jnp.arange(pad, dtype=jnp.int32) % (_NPAD - _N)
    dsts = jnp.concatenate([dst, pad_dst]).reshape(_NS, _NB, _CPB, _CH)
    zeros = jnp.zeros((_NPAD, _HD), jnp.float32)

    h = x
    h2 = jnp.stack([x[:, :_HD], x[:, _HD:]])
    outs = []
    for i in range(_L):
        agg2 = _sc_gather_segsum(h2, srcs, dsts, zeros)
        h, h2 = _tc_mlp_bn(h, agg2, W1[i], b1[i][None, :], W2[i],
                           b2[i][None, :], gamma[i][None, :], beta[i][None, :])
        outs.append(h)
    return jnp.concatenate(outs, axis=1)


# speedup vs baseline: 1.1650x; 1.1650x over previous
import functools
import jax
import jax.numpy as jnp
from jax import lax
from jax.experimental import pallas as pl
from jax.experimental.pallas import tpu as pltpu
from jax.experimental.pallas import tpu_sc as plsc

_N = 10000
_E = 320000
_D = 128
_HD = _D // 2
_L = 3
_BN_EPS = 1e-5

_NC = 2
_NS = 16
_CH = 128
_NB = 10
_CPB = 16
_CPT = _NB * _CPB
_EPT = _CPT * _CH
_EPAD = _NS * _EPT
_NPAD = 10112


def _sc_gather_segsum(h2, srcs, dsts, zeros):
    mesh = plsc.VectorSubcoreMesh(core_axis_name="c", subcore_axis_name="s")

    @functools.partial(
        pl.kernel,
        out_type=jax.ShapeDtypeStruct((_NC, _N, _HD), jnp.float32),
        mesh=mesh,
        scratch_types=[
            pltpu.VMEM_SHARED((_N, _HD), jnp.float32),
            pltpu.VMEM_SHARED((_NPAD, _HD), jnp.float32),
            pltpu.VMEM((2, _CPB, _CH), jnp.int32),
            pltpu.VMEM((2, _CPB, _CH), jnp.int32),
            pltpu.VMEM((2, _CH, _HD), jnp.float32),
            [pltpu.SemaphoreType.DMA] * 2,
            pltpu.SemaphoreType.DMA,
            pltpu.SemaphoreType.DMA,
        ],
    )
    def k(h2_hbm, srcs_hbm, dsts_hbm, zeros_hbm, out_hbm,
          h_s, agg_s, src_v, dst_v, rows_v, gsem, isem_s, isem_d):
        c = lax.axis_index("c")
        s = lax.axis_index("s")

        @pl.when(s < _NS - 1)
        def _():
            pltpu.sync_copy(h2_hbm.at[c, pl.ds(s * 624, 624)],
                            h_s.at[pl.ds(s * 624, 624)])

        @pl.when(s == _NS - 1)
        def _():
            pltpu.sync_copy(h2_hbm.at[c, pl.ds(9360, 640)],
                            h_s.at[pl.ds(9360, 640)])

        pltpu.sync_copy(zeros_hbm.at[pl.ds(s * 632, 632)],
                        agg_s.at[pl.ds(s * 632, 632)])
        pltpu.sync_copy(srcs_hbm.at[s, 0], src_v.at[0])
        pltpu.sync_copy(dsts_hbm.at[s, 0], dst_v.at[0])
        plsc.subcore_barrier()

        def fire_g(ksrc, j, q):
            pltpu.async_copy(h_s.at[ksrc.at[j]], rows_v.at[q], gsem[q])

        def wait_g(ksrc, j, q):
            pltpu.make_async_copy(h_s.at[ksrc.at[j]], rows_v.at[q],
                                  gsem[q]).wait()

        for k in range(_NB):
            b = k % 2
            nb = (k + 1) % 2
            ksrc = src_v.at[b]
            kdst = dst_v.at[b]
            if k + 1 < _NB:
                pltpu.async_copy(srcs_hbm.at[s, k + 1], src_v.at[nb], isem_s)
                pltpu.async_copy(dsts_hbm.at[s, k + 1], dst_v.at[nb], isem_d)

            fire_g(ksrc, 0, 0)

            def pair(t, _, ksrc=ksrc, kdst=kdst):
                fire_g(ksrc, 2 * t + 1, 1)
                wait_g(ksrc, 2 * t, 0)
                pltpu.sync_copy(rows_v.at[0], agg_s.at[kdst.at[2 * t]],
                                add=True)

                @pl.when(2 * t + 2 < _CPB)
                def _():
                    fire_g(ksrc, 2 * t + 2, 0)

                wait_g(ksrc, 2 * t + 1, 1)
                pltpu.sync_copy(rows_v.at[1], agg_s.at[kdst.at[2 * t + 1]],
                                add=True)
                return 0

            lax.fori_loop(0, _CPB // 2, pair, 0)

            if k + 1 < _NB:
                pltpu.make_async_copy(srcs_hbm.at[s, k + 1], src_v.at[nb],
                                      isem_s).wait()
                pltpu.make_async_copy(dsts_hbm.at[s, k + 1], dst_v.at[nb],
                                      isem_d).wait()
        plsc.subcore_barrier()

        @pl.when(s < _NS - 1)
        def _():
            pltpu.sync_copy(agg_s.at[pl.ds(s * 624, 624)],
                            out_hbm.at[c, pl.ds(s * 624, 624)])

        @pl.when(s == _NS - 1)
        def _():
            pltpu.sync_copy(agg_s.at[pl.ds(9360, 640)],
                            out_hbm.at[c, pl.ds(9360, 640)])

    return k(h2, srcs, dsts, zeros)


def _tc_mlp_bn(h, agg2, w1, b1, w2, b2, gm, bt):
    def body(h_ref, agg_ref, w1_ref, b1_ref, w2_ref, b2_ref, gm_ref, bt_ref,
             out_ref, out2_ref):
        z = h_ref[...] + jnp.concatenate([agg_ref[0], agg_ref[1]], axis=1)
        z = jnp.dot(z, w1_ref[...], preferred_element_type=jnp.float32)
        z = jnp.maximum(z + b1_ref[...], 0.0)
        z = jnp.dot(z, w2_ref[...], preferred_element_type=jnp.float32)
        z = jnp.maximum(z + b2_ref[...], 0.0)
        mean = jnp.mean(z, axis=0, keepdims=True)
        zc = z - mean
        var = jnp.mean(zc * zc, axis=0, keepdims=True)
        h_new = zc * lax.rsqrt(var + _BN_EPS) * gm_ref[...] + bt_ref[...]
        out_ref[...] = h_new
        out2_ref[0] = h_new[:, :_HD]
        out2_ref[1] = h_new[:, _HD:]

    return pl.pallas_call(
        body,
        out_shape=(jax.ShapeDtypeStruct((_N, _D), jnp.float32),
                   jax.ShapeDtypeStruct((_NC, _N, _HD), jnp.float32)),
    )(h, agg2, w1, b1, w2, b2, gm, bt)


def kernel(x, edge_index, batch, W1, b1, W2, b2, gamma, beta):
    src = edge_index[0]
    dst = edge_index[1]
    pad = _EPAD - _E
    srcs = jnp.concatenate([src, jnp.zeros((pad,), jnp.int32)]).reshape(
        _NS, _NB, _CPB, _CH)
    pad_dst = _N + jnp.arange(pad, dtype=jnp.int32) % (_NPAD - _N)
    dsts = jnp.concatenate([dst, pad_dst]).reshape(_NS, _NB, _CPB, _CH)
    zeros = jnp.zeros((_NPAD, _HD), jnp.float32)

    h = x
    h2 = jnp.stack([x[:, :_HD], x[:, _HD:]])
    outs = []
    for i in range(_L):
        agg2 = _sc_gather_segsum(h2, srcs, dsts, zeros)
        h, h2 = _tc_mlp_bn(h, agg2, W1[i], b1[i][None, :], W2[i],
                           b2[i][None, :], gamma[i][None, :], beta[i][None, :])
        outs.append(h)
    return jnp.concatenate(outs, axis=1)


# speedup vs baseline: 1.2196x; 1.0469x over previous
import functools
import jax
import jax.numpy as jnp
from jax import lax
from jax.experimental import pallas as pl
from jax.experimental.pallas import tpu as pltpu
from jax.experimental.pallas import tpu_sc as plsc

_N = 10000
_E = 320000
_D = 128
_HD = _D // 2
_L = 3
_BN_EPS = 1e-5

_NC = 2
_NS = 16
_CH = 128
_NB = 5
_CPB = 32
_CPT = _NB * _CPB
_EPT = _CPT * _CH
_EPAD = _NS * _EPT
_NPAD = 10112


def _sc_gather_segsum(h2, srcs, dsts, zeros):
    mesh = plsc.VectorSubcoreMesh(core_axis_name="c", subcore_axis_name="s")

    @functools.partial(
        pl.kernel,
        out_type=jax.ShapeDtypeStruct((_NC, _N, _HD), jnp.float32),
        mesh=mesh,
        scratch_types=[
            pltpu.VMEM_SHARED((_N, _HD), jnp.float32),
            pltpu.VMEM_SHARED((_NPAD, _HD), jnp.float32),
            pltpu.VMEM((2, _CPB, _CH), jnp.int32),
            pltpu.VMEM((2, _CPB, _CH), jnp.int32),
            pltpu.VMEM((2, _CH, _HD), jnp.float32),
            [pltpu.SemaphoreType.DMA] * 2,
            pltpu.SemaphoreType.DMA,
            pltpu.SemaphoreType.DMA,
        ],
    )
    def k(h2_hbm, srcs_hbm, dsts_hbm, zeros_hbm, out_hbm,
          h_s, agg_s, src_v, dst_v, rows_v, gsem, isem_s, isem_d):
        c = lax.axis_index("c")
        s = lax.axis_index("s")

        @pl.when(s < _NS - 1)
        def _():
            pltpu.sync_copy(h2_hbm.at[c, pl.ds(s * 624, 624)],
                            h_s.at[pl.ds(s * 624, 624)])

        @pl.when(s == _NS - 1)
        def _():
            pltpu.sync_copy(h2_hbm.at[c, pl.ds(9360, 640)],
                            h_s.at[pl.ds(9360, 640)])

        pltpu.sync_copy(zeros_hbm.at[pl.ds(s * 632, 632)],
                        agg_s.at[pl.ds(s * 632, 632)])
        pltpu.sync_copy(srcs_hbm.at[s, 0], src_v.at[0])
        pltpu.sync_copy(dsts_hbm.at[s, 0], dst_v.at[0])
        plsc.subcore_barrier()

        def fire_g(ksrc, j, q):
            pltpu.async_copy(h_s.at[ksrc.at[j]], rows_v.at[q], gsem[q])

        def wait_g(ksrc, j, q):
            pltpu.make_async_copy(h_s.at[ksrc.at[j]], rows_v.at[q],
                                  gsem[q]).wait()

        for k in range(_NB):
            b = k % 2
            nb = (k + 1) % 2
            ksrc = src_v.at[b]
            kdst = dst_v.at[b]
            if k + 1 < _NB:
                pltpu.async_copy(srcs_hbm.at[s, k + 1], src_v.at[nb], isem_s)
                pltpu.async_copy(dsts_hbm.at[s, k + 1], dst_v.at[nb], isem_d)

            fire_g(ksrc, 0, 0)

            def pair(t, _, ksrc=ksrc, kdst=kdst):
                fire_g(ksrc, 2 * t + 1, 1)
                wait_g(ksrc, 2 * t, 0)
                pltpu.sync_copy(rows_v.at[0], agg_s.at[kdst.at[2 * t]],
                                add=True)

                @pl.when(2 * t + 2 < _CPB)
                def _():
                    fire_g(ksrc, 2 * t + 2, 0)

                wait_g(ksrc, 2 * t + 1, 1)
                pltpu.sync_copy(rows_v.at[1], agg_s.at[kdst.at[2 * t + 1]],
                                add=True)
                return 0

            lax.fori_loop(0, _CPB // 2, pair, 0)

            if k + 1 < _NB:
                pltpu.make_async_copy(srcs_hbm.at[s, k + 1], src_v.at[nb],
                                      isem_s).wait()
                pltpu.make_async_copy(dsts_hbm.at[s, k + 1], dst_v.at[nb],
                                      isem_d).wait()
        plsc.subcore_barrier()

        @pl.when(s < _NS - 1)
        def _():
            pltpu.sync_copy(agg_s.at[pl.ds(s * 624, 624)],
                            out_hbm.at[c, pl.ds(s * 624, 624)])

        @pl.when(s == _NS - 1)
        def _():
            pltpu.sync_copy(agg_s.at[pl.ds(9360, 640)],
                            out_hbm.at[c, pl.ds(9360, 640)])

    return k(h2, srcs, dsts, zeros)


def _tc_mlp_bn(h, agg2, w1, b1, w2, b2, gm, bt):
    def body(h_ref, agg_ref, w1_ref, b1_ref, w2_ref, b2_ref, gm_ref, bt_ref,
             out_ref, out2_ref):
        z = h_ref[...] + jnp.concatenate([agg_ref[0], agg_ref[1]], axis=1)
        z = jnp.dot(z, w1_ref[...], preferred_element_type=jnp.float32)
        z = jnp.maximum(z + b1_ref[...], 0.0)
        z = jnp.dot(z, w2_ref[...], preferred_element_type=jnp.float32)
        z = jnp.maximum(z + b2_ref[...], 0.0)
        mean = jnp.mean(z, axis=0, keepdims=True)
        zc = z - mean
        var = jnp.mean(zc * zc, axis=0, keepdims=True)
        h_new = zc * lax.rsqrt(var + _BN_EPS) * gm_ref[...] + bt_ref[...]
        out_ref[...] = h_new
        out2_ref[0] = h_new[:, :_HD]
        out2_ref[1] = h_new[:, _HD:]

    return pl.pallas_call(
        body,
        out_shape=(jax.ShapeDtypeStruct((_N, _D), jnp.float32),
                   jax.ShapeDtypeStruct((_NC, _N, _HD), jnp.float32)),
    )(h, agg2, w1, b1, w2, b2, gm, bt)


def kernel(x, edge_index, batch, W1, b1, W2, b2, gamma, beta):
    src = edge_index[0]
    dst = edge_index[1]
    pad = _EPAD - _E
    srcs = jnp.concatenate([src, jnp.zeros((pad,), jnp.int32)]).reshape(
        _NS, _NB, _CPB, _CH)
    pad_dst = _N + jnp.arange(pad, dtype=jnp.int32) % (_NPAD - _N)
    dsts = jnp.concatenate([dst, pad_dst]).reshape(_NS, _NB, _CPB, _CH)
    zeros = jnp.zeros((_NPAD, _HD), jnp.float32)

    h = x
    h2 = jnp.stack([x[:, :_HD], x[:, _HD:]])
    outs = []
    for i in range(_L):
        agg2 = _sc_gather_segsum(h2, srcs, dsts, zeros)
        h, h2 = _tc_mlp_bn(h, agg2, W1[i], b1[i][None, :], W2[i],
                           b2[i][None, :], gamma[i][None, :], beta[i][None, :])
        outs.append(h)
    return jnp.concatenate(outs, axis=1)


# speedup vs baseline: 1.2323x; 1.0104x over previous
import functools
import jax
import jax.numpy as jnp
from jax import lax
from jax.experimental import pallas as pl
from jax.experimental.pallas import tpu as pltpu
from jax.experimental.pallas import tpu_sc as plsc

_N = 10000
_E = 320000
_D = 128
_HD = _D // 2
_L = 3
_BN_EPS = 1e-5

_NC = 2
_NS = 16
_CH = 128
_NB = 5
_CPB = 32
_CPT = _NB * _CPB
_EPT = _CPT * _CH
_EPAD = _NS * _EPT
_NPAD = 10112


def _sc_gather_segsum(h2, srcs, dsts, zeros):
    mesh = plsc.VectorSubcoreMesh(core_axis_name="c", subcore_axis_name="s")

    @functools.partial(
        pl.kernel,
        out_type=jax.ShapeDtypeStruct((_NC, _N, _HD), jnp.float32),
        mesh=mesh,
        scratch_types=[
            pltpu.VMEM_SHARED((_N, _HD), jnp.float32),
            pltpu.VMEM_SHARED((_NPAD, _HD), jnp.float32),
            pltpu.VMEM((2, _CPB, _CH), jnp.int32),
            pltpu.VMEM((2, _CPB, _CH), jnp.int32),
            pltpu.VMEM((2, _CH, _HD), jnp.float32),
            [pltpu.SemaphoreType.DMA] * 2,
            pltpu.SemaphoreType.DMA,
            pltpu.SemaphoreType.DMA,
        ],
    )
    def k(h2_hbm, srcs_hbm, dsts_hbm, zeros_hbm, out_hbm,
          h_s, agg_s, src_v, dst_v, rows_v, gsem, isem_s, isem_d):
        c = lax.axis_index("c")
        s = lax.axis_index("s")

        @pl.when(s < _NS - 1)
        def _():
            pltpu.sync_copy(h2_hbm.at[c, pl.ds(s * 624, 624)],
                            h_s.at[pl.ds(s * 624, 624)])

        @pl.when(s == _NS - 1)
        def _():
            pltpu.sync_copy(h2_hbm.at[c, pl.ds(9360, 640)],
                            h_s.at[pl.ds(9360, 640)])

        pltpu.sync_copy(zeros_hbm.at[pl.ds(s * 632, 632)],
                        agg_s.at[pl.ds(s * 632, 632)])
        pltpu.sync_copy(srcs_hbm.at[s, 0], src_v.at[0])
        pltpu.sync_copy(dsts_hbm.at[s, 0], dst_v.at[0])
        plsc.subcore_barrier()

        def fire_g(ksrc, j, q):
            pltpu.async_copy(h_s.at[ksrc.at[j]], rows_v.at[q], gsem[q])

        def wait_g(ksrc, j, q):
            pltpu.make_async_copy(h_s.at[ksrc.at[j]], rows_v.at[q],
                                  gsem[q]).wait()

        for k in range(_NB):
            b = k % 2
            nb = (k + 1) % 2
            ksrc = src_v.at[b]
            kdst = dst_v.at[b]
            if k + 1 < _NB:
                pltpu.async_copy(srcs_hbm.at[s, k + 1], src_v.at[nb], isem_s)
                pltpu.async_copy(dsts_hbm.at[s, k + 1], dst_v.at[nb], isem_d)

            fire_g(ksrc, 0, 0)

            def pair(t, _, ksrc=ksrc, kdst=kdst):
                fire_g(ksrc, 2 * t + 1, 1)
                wait_g(ksrc, 2 * t, 0)
                pltpu.sync_copy(rows_v.at[0], agg_s.at[kdst.at[2 * t]],
                                add=True)

                @pl.when(2 * t + 2 < _CPB)
                def _():
                    fire_g(ksrc, 2 * t + 2, 0)

                wait_g(ksrc, 2 * t + 1, 1)
                pltpu.sync_copy(rows_v.at[1], agg_s.at[kdst.at[2 * t + 1]],
                                add=True)
                return 0

            lax.fori_loop(0, _CPB // 2, pair, 0)

            if k + 1 < _NB:
                pltpu.make_async_copy(srcs_hbm.at[s, k + 1], src_v.at[nb],
                                      isem_s).wait()
                pltpu.make_async_copy(dsts_hbm.at[s, k + 1], dst_v.at[nb],
                                      isem_d).wait()
        plsc.subcore_barrier()

        @pl.when(s < _NS - 1)
        def _():
            pltpu.sync_copy(agg_s.at[pl.ds(s * 624, 624)],
                            out_hbm.at[c, pl.ds(s * 624, 624)])

        @pl.when(s == _NS - 1)
        def _():
            pltpu.sync_copy(agg_s.at[pl.ds(9360, 640)],
                            out_hbm.at[c, pl.ds(9360, 640)])

    return k(h2, srcs, dsts, zeros)


def _mlp_bn(z, w1, b1, w2, b2, gm, bt):
    z = jnp.dot(z, w1, preferred_element_type=jnp.float32)
    z = jnp.maximum(z + b1, 0.0)
    z = jnp.dot(z, w2, preferred_element_type=jnp.float32)
    z = jnp.maximum(z + b2, 0.0)
    mean = jnp.mean(z, axis=0, keepdims=True)
    zc = z - mean
    var = jnp.mean(zc * zc, axis=0, keepdims=True)
    return zc * lax.rsqrt(var + _BN_EPS) * gm + bt


def _tc_mlp_bn(h, agg2, w1, b1, w2, b2, gm, bt):
    def body(h_ref, agg_ref, w1_ref, b1_ref, w2_ref, b2_ref, gm_ref, bt_ref,
             out_ref, out2_ref):
        z = h_ref[...] + jnp.concatenate([agg_ref[0], agg_ref[1]], axis=1)
        h_new = _mlp_bn(z, w1_ref[...], b1_ref[...], w2_ref[...], b2_ref[...],
                        gm_ref[...], bt_ref[...])
        out_ref[...] = h_new
        out2_ref[0] = h_new[:, :_HD]
        out2_ref[1] = h_new[:, _HD:]

    return pl.pallas_call(
        body,
        out_shape=(jax.ShapeDtypeStruct((_N, _D), jnp.float32),
                   jax.ShapeDtypeStruct((_NC, _N, _HD), jnp.float32)),
    )(h, agg2, w1, b1, w2, b2, gm, bt)


def _tc_mlp_bn_final(h1, h2, h, agg2, w1, b1, w2, b2, gm, bt):
    def body(h1_ref, h2_ref, h_ref, agg_ref, w1_ref, b1_ref, w2_ref, b2_ref,
             gm_ref, bt_ref, out_ref):
        z = h_ref[...] + jnp.concatenate([agg_ref[0], agg_ref[1]], axis=1)
        h_new = _mlp_bn(z, w1_ref[...], b1_ref[...], w2_ref[...], b2_ref[...],
                        gm_ref[...], bt_ref[...])
        out_ref[:, :_D] = h1_ref[...]
        out_ref[:, _D:2 * _D] = h2_ref[...]
        out_ref[:, 2 * _D:] = h_new

    return pl.pallas_call(
        body,
        out_shape=jax.ShapeDtypeStruct((_N, _L * _D), jnp.float32),
    )(h1, h2, h, agg2, w1, b1, w2, b2, gm, bt)


def kernel(x, edge_index, batch, W1, b1, W2, b2, gamma, beta):
    src = edge_index[0]
    dst = edge_index[1]
    pad = _EPAD - _E
    srcs = jnp.concatenate([src, jnp.zeros((pad,), jnp.int32)]).reshape(
        _NS, _NB, _CPB, _CH)
    pad_dst = _N + jnp.arange(pad, dtype=jnp.int32) % (_NPAD - _N)
    dsts = jnp.concatenate([dst, pad_dst]).reshape(_NS, _NB, _CPB, _CH)
    zeros = jnp.zeros((_NPAD, _HD), jnp.float32)

    h = x
    h2 = jnp.stack([x[:, :_HD], x[:, _HD:]])
    outs = []
    for i in range(_L - 1):
        agg2 = _sc_gather_segsum(h2, srcs, dsts, zeros)
        h, h2 = _tc_mlp_bn(h, agg2, W1[i], b1[i][None, :], W2[i],
                           b2[i][None, :], gamma[i][None, :], beta[i][None, :])
        outs.append(h)
    agg2 = _sc_gather_segsum(h2, srcs, dsts, zeros)
    i = _L - 1
    return _tc_mlp_bn_final(outs[0], outs[1], h, agg2, W1[i], b1[i][None, :],
                            W2[i], b2[i][None, :], gamma[i][None, :],
                            beta[i][None, :])
